# baseline (speedup 1.0000x reference)
import functools

import jax
import jax.numpy as jnp
from jax import lax
from jax.experimental import pallas as pl
from jax.experimental.pallas import tpu as pltpu
from jax.experimental.pallas import tpu_sc as plsc

F = 128
R = 20
CUTOFF = 5.0

N_CORES = 2
N_SUBCORES = 16
N_WORKERS = N_CORES * N_SUBCORES

GC = 40
SCC = 40
BE = 1280
NCH = 5
BN = 512
HOP = 128
IB = 20
NB = 4


def _mlp_call(ns_pad, W1, b1r, W2, b2r):
    npad = ns_pad.shape[0]

    def body(ns_ref, w1_ref, b1_ref, w2_ref, b2_ref, out_ref):
        x = ns_ref[...]
        h = jnp.dot(x, w1_ref[...], preferred_element_type=jnp.float32,
                    precision=lax.Precision.HIGHEST) + b1_ref[...]
        h = h * jax.nn.sigmoid(h)
        out_ref[...] = jnp.dot(h, w2_ref[...], preferred_element_type=jnp.float32,
                               precision=lax.Precision.HIGHEST) + b2_ref[...]

    return pl.pallas_call(
        body,
        grid=(npad // BN,),
        in_specs=[
            pl.BlockSpec((BN, F), lambda i: (i, 0)),
            pl.BlockSpec((F, F), lambda i: (0, 0)),
            pl.BlockSpec((1, F), lambda i: (0, 0)),
            pl.BlockSpec((F, 3 * F), lambda i: (0, 0)),
            pl.BlockSpec((1, 3 * F), lambda i: (0, 0)),
        ],
        out_specs=pl.BlockSpec((BN, 3 * F), lambda i: (i, 0)),
        out_shape=jax.ShapeDtypeStruct((npad, 3 * F), jnp.float32),
    )(ns_pad, W1, b1r, W2, b2r)


def _gather_call(atom, nvr, src3, e_pad):
    epw = e_pad // N_WORKERS
    n_chunks = epw // GC
    mesh = plsc.VectorSubcoreMesh(core_axis_name="c", subcore_axis_name="s")

    @functools.partial(
        pl.kernel,
        out_type=(
            jax.ShapeDtypeStruct((e_pad, 3 * F), jnp.float32),
            jax.ShapeDtypeStruct((e_pad, 3 * F), jnp.float32),
        ),
        mesh=mesh,
        scratch_types=[
            pltpu.VMEM((n_chunks, GC), jnp.int32),
            pltpu.VMEM((2 * GC, 3 * F), jnp.float32),
            pltpu.VMEM((2 * GC, 3 * F), jnp.float32),
            pltpu.SemaphoreType.DMA,
            pltpu.SemaphoreType.DMA,
            pltpu.SemaphoreType.DMA,
            pltpu.SemaphoreType.DMA,
        ],
    )
    def gather_k(atom_hbm, nvr_hbm, src3_hbm, ga_hbm, gv_hbm,
                 idx_v, buf_a, buf_v, sa0, sa1, sv0, sv1):
        c = lax.axis_index("c")
        s = lax.axis_index("s")
        wid = s * N_CORES + c
        sa = (sa0, sa1)
        sv = (sv0, sv1)
        pltpu.sync_copy(src3_hbm.at[wid], idx_v)

        def issue(chunk, slot):
            bsl = pl.ds(slot * GC, GC)
            pltpu.async_copy(atom_hbm.at[idx_v.at[chunk]], buf_a.at[bsl], sa[slot])
            pltpu.async_copy(nvr_hbm.at[idx_v.at[chunk]], buf_v.at[bsl], sv[slot])

        issue(0, 0)

        def body(g, carry):
            for j in range(2):
                chunk = g * 2 + j
                nslot = (j + 1) % 2

                @pl.when(chunk + 1 < n_chunks)
                def _():
                    issue(chunk + 1, nslot)

                bsl = pl.ds(j * GC, GC)
                pltpu.make_async_copy(
                    atom_hbm.at[idx_v.at[chunk]], buf_a.at[bsl], sa[j]).wait()
                pltpu.make_async_copy(
                    nvr_hbm.at[idx_v.at[chunk]], buf_v.at[bsl], sv[j]).wait()
                base = (wid * n_chunks + chunk) * GC
                pltpu.sync_copy(buf_a.at[bsl], ga_hbm.at[pl.ds(base, GC)])
                pltpu.sync_copy(buf_v.at[bsl], gv_hbm.at[pl.ds(base, GC)])
            return carry

        lax.fori_loop(0, n_chunks // 2, body, 0)

    return gather_k(atom, nvr, src3)


def _sinc_call(dist2):
    rows = dist2.shape[0]
    bd = rows

    def body(d_ref, out_ref):
        d = d_ref[...]
        th = d * (jnp.pi / CUTOFF)
        s1 = jnp.sin(th)
        c1 = jnp.cos(th)
        cc = jnp.where(d < CUTOFF, 0.5 * (c1 + 1.0), 0.0)
        inv = cc / d
        twoc = 2.0 * c1
        out_ref[0] = s1 * inv
        s2 = twoc * s1
        out_ref[1] = s2 * inv
        prev2, prev1 = s1, s2
        for k in range(3, R + 1):
            sk = twoc * prev1 - prev2
            out_ref[k - 1] = sk * inv
            prev2, prev1 = prev1, sk
        out_ref[R] = cc
        zero = jnp.zeros_like(d)
        for k in range(R + 1, R + 4):
            out_ref[k] = zero

    return pl.pallas_call(
        body,
        grid=(rows // bd,),
        in_specs=[pl.BlockSpec((bd, F), lambda i: (i, 0))],
        out_specs=pl.BlockSpec((R + 4, bd, F), lambda i: (0, i, 0)),
        out_shape=jax.ShapeDtypeStruct((R + 4, rows, F), jnp.float32),
    )(dist2)


def _combine_call(adj, feats, ga, gv, wrbf_ext, e_pad):
    def body(adj_ref, feats_ref, ga_ref, gv_ref, wrbf_ref, out_ref):
        adj_b = adj_ref[...]
        dist = adj_b[:, 5:6]
        rvec = adj_b[:, 2:5]
        rbf = lax.dot_general(feats_ref[...], wrbf_ref[...],
                              dimension_numbers=(((0,), (0,)), ((), ())),
                              preferred_element_type=jnp.float32,
                              precision=lax.Precision.HIGHEST)
        pre = ga_ref[...] * rbf
        s1 = pre[:, :F]
        s2 = pre[:, F:2 * F]
        s3 = pre[:, 2 * F:]
        gv_b = gv_ref[...]
        rhat = rvec / dist
        out_ref[0] = s2
        out_ref[1] = gv_b[:, :F] * s1 + s3 * rhat[:, 0:1]
        out_ref[2] = gv_b[:, F:2 * F] * s1 + s3 * rhat[:, 1:2]
        out_ref[3] = gv_b[:, 2 * F:] * s1 + s3 * rhat[:, 2:3]

    return pl.pallas_call(
        body,
        grid=(e_pad // BE,),
        in_specs=[
            pl.BlockSpec((BE, 6), lambda i: (i, 0)),
            pl.BlockSpec((R + 4, BE), lambda i: (0, i)),
            pl.BlockSpec((BE, 3 * F), lambda i: (i, 0)),
            pl.BlockSpec((BE, 3 * F), lambda i: (i, 0)),
            pl.BlockSpec((R + 4, 3 * F), lambda i: (0, 0)),
        ],
        out_specs=pl.BlockSpec((4, BE, F), lambda i: (0, i, 0)),
        out_shape=jax.ShapeDtypeStruct((4, e_pad, F), jnp.float32),
    )(adj, feats, ga, gv, wrbf_ext)


def _scatter_call(msgs_list, base, dst5, n_pad, e_chunk):
    ept = e_chunk // N_SUBCORES
    n_chunks = ept // SCC
    n_groups = n_chunks // IB
    rows_per_tile = n_pad // N_SUBCORES
    n_hops = rows_per_tile // HOP
    mesh = plsc.VectorSubcoreMesh(core_axis_name="c", subcore_axis_name="s")

    @functools.partial(
        pl.kernel,
        out_type=jax.ShapeDtypeStruct((4, n_pad, F), jnp.float32),
        mesh=mesh,
        scratch_types=[
            pltpu.VMEM((IB, SCC), jnp.int32),
            pltpu.VMEM((NB * SCC, F), jnp.float32),
            pltpu.VMEM_SHARED((n_pad, F), jnp.float32),
            pltpu.SemaphoreType.DMA,
            pltpu.SemaphoreType.DMA,
            pltpu.SemaphoreType.DMA,
            pltpu.SemaphoreType.DMA,
            pltpu.SemaphoreType.DMA,
            pltpu.SemaphoreType.DMA,
            pltpu.SemaphoreType.DMA,
            pltpu.SemaphoreType.DMA,
        ],
    )
    def scatter_k(m0, m1, m2, m3, m4, base_hbm, dst5_hbm, out_hbm,
                  idx_v, upd_v, acc,
                  ls0, ls1, ls2, ls3, ss0, ss1, ss2, ss3):
        msgs_refs = (m0, m1, m2, m3, m4)
        c = lax.axis_index("c")
        s = lax.axis_index("s")
        ls = (ls0, ls1, ls2, ls3)
        ss = (ss0, ss1, ss2, ss3)
        row0 = s * rows_per_tile
        stage = upd_v.at[pl.ds(0, HOP)]
        last_slot = (IB - 1) % NB

        def drain_last():
            pltpu.make_async_copy(
                upd_v.at[pl.ds(last_slot * SCC, SCC)],
                acc.at[idx_v.at[0]], ss[last_slot]).wait()

        for k in range(2):
            p = 2 * c + k
            def init_hop(h, carry):
                pltpu.sync_copy(base_hbm.at[p, pl.ds(row0 + h * HOP, HOP)],
                                stage)
                pltpu.sync_copy(stage, acc.at[pl.ds(row0 + h * HOP, HOP)])
                return carry

            lax.fori_loop(0, n_hops, init_hop, 0)
            plsc.subcore_barrier()

            for m in range(NCH):
                mh = msgs_refs[m]

                def load_src(chunk):
                    return mh.at[p, pl.ds(s * ept + chunk * SCC, SCC)]

                def issue_load(chunk, slot):
                    pltpu.async_copy(load_src(chunk),
                                     upd_v.at[pl.ds(slot * SCC, SCC)],
                                     ls[slot])

                for slot in range(NB - 1):
                    issue_load(slot, slot)

                def sc_group(g, carry):
                    @pl.when(g >= 1)
                    def _():
                        drain_last()

                    pltpu.sync_copy(dst5_hbm.at[m, s, g], idx_v)
                    for j in range(IB):
                        chunk = g * IB + j
                        slot = j % NB
                        bsl = pl.ds(slot * SCC, SCC)
                        pltpu.make_async_copy(load_src(chunk), upd_v.at[bsl],
                                              ls[slot]).wait()
                        pltpu.async_copy(upd_v.at[bsl], acc.at[idx_v.at[j]],
                                         ss[slot], add=True)
                        pslot = (j - 1) % NB
                        if j >= 1:
                            pltpu.make_async_copy(
                                upd_v.at[pl.ds(pslot * SCC, SCC)],
                                acc.at[idx_v.at[j]], ss[pslot]).wait()

                        @pl.when(chunk + NB - 1 < n_chunks)
                        def _():
                            issue_load(chunk + NB - 1, pslot)
                    return carry

                lax.fori_loop(0, n_groups, sc_group, 0)
                drain_last()

            plsc.subcore_barrier()

            def wb_hop(h, carry):
                pltpu.sync_copy(acc.at[pl.ds(row0 + h * HOP, HOP)], stage)
                pltpu.sync_copy(stage,
                                out_hbm.at[p, pl.ds(row0 + h * HOP, HOP)])
                return carry

            lax.fori_loop(0, n_hops, wb_hop, 0)
            plsc.subcore_barrier()

    return scatter_k(*msgs_list, base, dst5)


def kernel(node_scalar, node_vector, adj_matrix, W1, b1, W2, b2, Wrbf, brbf):
    n, f = node_scalar.shape
    e = adj_matrix.shape[0]
    assert f == F
    n_pad = ((n + N_SUBCORES * HOP - 1) // (N_SUBCORES * HOP)) * (N_SUBCORES * HOP)
    e_chunk = e // NCH
    assert e_chunk * NCH == e and e_chunk % (N_WORKERS * GC) == 0
    assert e_chunk % BE == 0 and e_chunk % (N_SUBCORES * SCC * IB) == 0

    src_i = adj_matrix[:, 1].astype(jnp.int32)
    dst_i = adj_matrix[:, 0].astype(jnp.int32)

    ns_pad = jnp.pad(node_scalar, ((0, n_pad - n), (0, 0)))
    atom = _mlp_call(ns_pad, W1, b1.reshape(1, F), W2, b2.reshape(1, 3 * F))

    nvr = jnp.swapaxes(node_vector, 1, 2).reshape(n, 3 * F)
    src5 = src_i.reshape(NCH, N_WORKERS, -1, GC)
    wrbf_ext = jnp.concatenate(
        [Wrbf, brbf[None, :], jnp.zeros((3, 3 * F), jnp.float32)], axis=0)
    msgs_list = []
    for ch in range(NCH):
        adj_ch = lax.dynamic_slice_in_dim(adj_matrix, ch * e_chunk, e_chunk, 0)
        ga, gv = _gather_call(atom, nvr, src5[ch], e_chunk)
        dist2 = adj_ch[:, 5].reshape(-1, F)
        feats = _sinc_call(dist2).reshape(R + 4, e_chunk)
        msgs_list.append(_combine_call(adj_ch, feats, ga, gv, wrbf_ext, e_chunk))

    nvt = jnp.moveaxis(node_vector, -1, 0)
    base = jnp.concatenate([node_scalar[None], nvt], axis=0)
    base = jnp.pad(base, ((0, 0), (0, n_pad - n), (0, 0)))
    dst5 = dst_i.reshape(NCH, N_SUBCORES, -1, IB, SCC)
    out = _scatter_call(msgs_list, base, dst5, n_pad, e_chunk)

    out_s = out[0, :n]
    out_v = jnp.stack([out[1, :n], out[2, :n], out[3, :n]], axis=-1)
    return out_s, out_v


# speedup vs baseline: 1.0048x; 1.0048x over previous
import functools

import jax
import jax.numpy as jnp
from jax import lax
from jax.experimental import pallas as pl
from jax.experimental.pallas import tpu as pltpu
from jax.experimental.pallas import tpu_sc as plsc

F = 128
R = 20
CUTOFF = 5.0

N_CORES = 2
N_SUBCORES = 16
N_WORKERS = N_CORES * N_SUBCORES

GC = 40
SCC = 40
BE = 1280
NCH = 2
BN = 512
HOP = 128
IB = 10
NB = 5


def _mlp_call(ns_pad, W1, b1r, W2, b2r):
    npad = ns_pad.shape[0]

    def body(ns_ref, w1_ref, b1_ref, w2_ref, b2_ref, out_ref):
        x = ns_ref[...]
        h = jnp.dot(x, w1_ref[...], preferred_element_type=jnp.float32,
                    precision=lax.Precision.HIGHEST) + b1_ref[...]
        h = h * jax.nn.sigmoid(h)
        out_ref[...] = jnp.dot(h, w2_ref[...], preferred_element_type=jnp.float32,
                               precision=lax.Precision.HIGHEST) + b2_ref[...]

    return pl.pallas_call(
        body,
        grid=(npad // BN,),
        in_specs=[
            pl.BlockSpec((BN, F), lambda i: (i, 0)),
            pl.BlockSpec((F, F), lambda i: (0, 0)),
            pl.BlockSpec((1, F), lambda i: (0, 0)),
            pl.BlockSpec((F, 3 * F), lambda i: (0, 0)),
            pl.BlockSpec((1, 3 * F), lambda i: (0, 0)),
        ],
        out_specs=pl.BlockSpec((BN, 3 * F), lambda i: (i, 0)),
        out_shape=jax.ShapeDtypeStruct((npad, 3 * F), jnp.float32),
    )(ns_pad, W1, b1r, W2, b2r)


def _gather_call(atom, nvr, src3, e_pad):
    epw = e_pad // N_WORKERS
    n_chunks = epw // GC
    mesh = plsc.VectorSubcoreMesh(core_axis_name="c", subcore_axis_name="s")

    @functools.partial(
        pl.kernel,
        out_type=(
            jax.ShapeDtypeStruct((e_pad // GC, GC, 3 * F), jnp.float32),
            jax.ShapeDtypeStruct((e_pad // GC, GC, 3 * F), jnp.float32),
        ),
        mesh=mesh,
        scratch_types=[
            pltpu.VMEM((n_chunks, GC), jnp.int32),
            pltpu.VMEM((2, GC, 3 * F), jnp.float32),
            pltpu.VMEM((2, GC, 3 * F), jnp.float32),
            pltpu.SemaphoreType.DMA,
            pltpu.SemaphoreType.DMA,
            pltpu.SemaphoreType.DMA,
            pltpu.SemaphoreType.DMA,
        ],
    )
    def gather_k(atom_hbm, nvr_hbm, src3_hbm, ga_hbm, gv_hbm,
                 idx_v, buf_a, buf_v, sa0, sa1, sv0, sv1):
        c = lax.axis_index("c")
        s = lax.axis_index("s")
        wid = s * N_CORES + c
        sa = (sa0, sa1)
        sv = (sv0, sv1)
        pltpu.sync_copy(src3_hbm.at[wid], idx_v)

        def issue(chunk, slot):
            pltpu.async_copy(atom_hbm.at[idx_v.at[chunk]], buf_a.at[slot], sa[slot])
            pltpu.async_copy(nvr_hbm.at[idx_v.at[chunk]], buf_v.at[slot], sv[slot])

        issue(0, 0)

        def body(g, carry):
            for j in range(2):
                chunk = g * 2 + j
                nslot = (j + 1) % 2

                @pl.when(chunk + 1 < n_chunks)
                def _():
                    issue(chunk + 1, nslot)

                pltpu.make_async_copy(
                    atom_hbm.at[idx_v.at[chunk]], buf_a.at[j], sa[j]).wait()
                pltpu.make_async_copy(
                    nvr_hbm.at[idx_v.at[chunk]], buf_v.at[j], sv[j]).wait()
                row = wid * n_chunks + chunk
                pltpu.sync_copy(buf_a.at[j], ga_hbm.at[row])
                pltpu.sync_copy(buf_v.at[j], gv_hbm.at[row])
            return carry

        lax.fori_loop(0, n_chunks // 2, body, 0)

        if n_chunks % 2 == 1:
            chunk = n_chunks - 1
            pltpu.make_async_copy(
                atom_hbm.at[idx_v.at[chunk]], buf_a.at[0], sa[0]).wait()
            pltpu.make_async_copy(
                nvr_hbm.at[idx_v.at[chunk]], buf_v.at[0], sv[0]).wait()
            row = wid * n_chunks + chunk
            pltpu.sync_copy(buf_a.at[0], ga_hbm.at[row])
            pltpu.sync_copy(buf_v.at[0], gv_hbm.at[row])

    return gather_k(atom, nvr, src3)


def _sinc_call(dist2):
    rows = dist2.shape[0]
    bd = rows

    def body(d_ref, out_ref):
        d = d_ref[...]
        th = d * (jnp.pi / CUTOFF)
        s1 = jnp.sin(th)
        c1 = jnp.cos(th)
        cc = jnp.where(d < CUTOFF, 0.5 * (c1 + 1.0), 0.0)
        inv = cc / d
        twoc = 2.0 * c1
        out_ref[0] = s1 * inv
        s2 = twoc * s1
        out_ref[1] = s2 * inv
        prev2, prev1 = s1, s2
        for k in range(3, R + 1):
            sk = twoc * prev1 - prev2
            out_ref[k - 1] = sk * inv
            prev2, prev1 = prev1, sk
        out_ref[R] = cc
        zero = jnp.zeros_like(d)
        for k in range(R + 1, R + 4):
            out_ref[k] = zero

    return pl.pallas_call(
        body,
        grid=(rows // bd,),
        in_specs=[pl.BlockSpec((bd, F), lambda i: (i, 0))],
        out_specs=pl.BlockSpec((R + 4, bd, F), lambda i: (0, i, 0)),
        out_shape=jax.ShapeDtypeStruct((R + 4, rows, F), jnp.float32),
    )(dist2)


def _combine_call(adj, feats, ga, gv, wrbf_ext, e_pad):
    def body(adj_ref, feats_ref, ga_ref, gv_ref, wrbf_ref, out_ref):
        adj_b = adj_ref[...]
        dist = adj_b[:, 5:6]
        rvec = adj_b[:, 2:5]
        rbf = lax.dot_general(feats_ref[...], wrbf_ref[...],
                              dimension_numbers=(((0,), (0,)), ((), ())),
                              preferred_element_type=jnp.float32,
                              precision=lax.Precision.HIGHEST)
        pre = ga_ref[...].astype(jnp.float32) * rbf
        s1 = pre[:, :F]
        s2 = pre[:, F:2 * F]
        s3 = pre[:, 2 * F:]
        gv_b = gv_ref[...].astype(jnp.float32)
        rhat = rvec / dist
        out_ref[0] = s2
        out_ref[1] = gv_b[:, :F] * s1 + s3 * rhat[:, 0:1]
        out_ref[2] = gv_b[:, F:2 * F] * s1 + s3 * rhat[:, 1:2]
        out_ref[3] = gv_b[:, 2 * F:] * s1 + s3 * rhat[:, 2:3]

    return pl.pallas_call(
        body,
        grid=(e_pad // BE,),
        in_specs=[
            pl.BlockSpec((BE, 6), lambda i: (i, 0)),
            pl.BlockSpec((R + 4, BE), lambda i: (0, i)),
            pl.BlockSpec((BE, 3 * F), lambda i: (i, 0)),
            pl.BlockSpec((BE, 3 * F), lambda i: (i, 0)),
            pl.BlockSpec((R + 4, 3 * F), lambda i: (0, 0)),
        ],
        out_specs=pl.BlockSpec((4, BE, F), lambda i: (0, i, 0)),
        out_shape=jax.ShapeDtypeStruct((4, e_pad, F), jnp.float32),
    )(adj, feats, ga, gv, wrbf_ext)


def _scatter_call(msgs_list, base, dst5, n_pad, e_chunk):
    ept = e_chunk // N_SUBCORES
    n_chunks = ept // SCC
    n_groups = n_chunks // IB
    rows_per_tile = n_pad // N_SUBCORES
    n_hops = rows_per_tile // HOP
    mesh = plsc.VectorSubcoreMesh(core_axis_name="c", subcore_axis_name="s")

    @functools.partial(
        pl.kernel,
        out_type=jax.ShapeDtypeStruct((4, n_pad, F), jnp.float32),
        mesh=mesh,
        scratch_types=[
            pltpu.VMEM((IB, SCC), jnp.int32),
            pltpu.VMEM((NB * SCC, F), jnp.float32),
            pltpu.VMEM_SHARED((n_pad, F), jnp.float32),
            pltpu.SemaphoreType.DMA,
            pltpu.SemaphoreType.DMA,
            pltpu.SemaphoreType.DMA,
            pltpu.SemaphoreType.DMA,
            pltpu.SemaphoreType.DMA,
            pltpu.SemaphoreType.DMA,
            pltpu.SemaphoreType.DMA,
            pltpu.SemaphoreType.DMA,
            pltpu.SemaphoreType.DMA,
            pltpu.SemaphoreType.DMA,
        ],
    )
    def scatter_k(m0, m1, base_hbm, dst5_hbm, out_hbm,
                  idx_v, upd_v, acc,
                  ls0, ls1, ls2, ls3, ls4, ss0, ss1, ss2, ss3, ss4):
        msgs_refs = (m0, m1)
        c = lax.axis_index("c")
        s = lax.axis_index("s")
        ls = (ls0, ls1, ls2, ls3, ls4)
        ss = (ss0, ss1, ss2, ss3, ss4)
        row0 = s * rows_per_tile
        stage = upd_v.at[pl.ds(0, HOP)]
        last_slot = (IB - 1) % NB

        def drain_last():
            pltpu.make_async_copy(
                upd_v.at[pl.ds(last_slot * SCC, SCC)],
                acc.at[idx_v.at[0]], ss[last_slot]).wait()

        for k in range(2):
            p = 2 * c + k
            def init_hop(h, carry):
                pltpu.sync_copy(base_hbm.at[p, pl.ds(row0 + h * HOP, HOP)],
                                stage)
                pltpu.sync_copy(stage, acc.at[pl.ds(row0 + h * HOP, HOP)])
                return carry

            lax.fori_loop(0, n_hops, init_hop, 0)
            plsc.subcore_barrier()

            for m in range(NCH):
                mh = msgs_refs[m]

                def load_src(chunk):
                    return mh.at[p, pl.ds(s * ept + chunk * SCC, SCC)]

                def issue_load(chunk, slot):
                    pltpu.async_copy(load_src(chunk),
                                     upd_v.at[pl.ds(slot * SCC, SCC)],
                                     ls[slot])

                for slot in range(NB - 1):
                    issue_load(slot, slot)

                def sc_group(g, carry):
                    @pl.when(g >= 1)
                    def _():
                        drain_last()

                    pltpu.sync_copy(dst5_hbm.at[m, s, g], idx_v)
                    for j in range(IB):
                        chunk = g * IB + j
                        slot = j % NB
                        bsl = pl.ds(slot * SCC, SCC)
                        pltpu.make_async_copy(load_src(chunk), upd_v.at[bsl],
                                              ls[slot]).wait()
                        pltpu.async_copy(upd_v.at[bsl], acc.at[idx_v.at[j]],
                                         ss[slot], add=True)
                        pslot = (j - 1) % NB
                        if j >= 1:
                            pltpu.make_async_copy(
                                upd_v.at[pl.ds(pslot * SCC, SCC)],
                                acc.at[idx_v.at[j]], ss[pslot]).wait()

                        @pl.when(chunk + NB - 1 < n_chunks)
                        def _():
                            issue_load(chunk + NB - 1, pslot)
                    return carry

                lax.fori_loop(0, n_groups, sc_group, 0)
                drain_last()

            plsc.subcore_barrier()

            def wb_hop(h, carry):
                pltpu.sync_copy(acc.at[pl.ds(row0 + h * HOP, HOP)], stage)
                pltpu.sync_copy(stage,
                                out_hbm.at[p, pl.ds(row0 + h * HOP, HOP)])
                return carry

            lax.fori_loop(0, n_hops, wb_hop, 0)
            plsc.subcore_barrier()

    return scatter_k(*msgs_list, base, dst5)


def kernel(node_scalar, node_vector, adj_matrix, W1, b1, W2, b2, Wrbf, brbf):
    n, f = node_scalar.shape
    e = adj_matrix.shape[0]
    assert f == F
    n_pad = ((n + N_SUBCORES * HOP - 1) // (N_SUBCORES * HOP)) * (N_SUBCORES * HOP)
    e_chunk = e // NCH
    assert e_chunk * NCH == e and e_chunk % (N_WORKERS * GC) == 0
    assert e_chunk % BE == 0 and e_chunk % (N_SUBCORES * SCC * IB) == 0

    src_i = adj_matrix[:, 1].astype(jnp.int32)
    dst_i = adj_matrix[:, 0].astype(jnp.int32)

    ns_pad = jnp.pad(node_scalar, ((0, n_pad - n), (0, 0)))
    atom = _mlp_call(ns_pad, W1, b1.reshape(1, F), W2, b2.reshape(1, 3 * F))

    nvr = jnp.swapaxes(node_vector, 1, 2).reshape(n, 3 * F)
    src5 = src_i.reshape(NCH, N_WORKERS, -1, GC)
    wrbf_ext = jnp.concatenate(
        [Wrbf, brbf[None, :], jnp.zeros((3, 3 * F), jnp.float32)], axis=0)
    msgs_list = []
    for ch in range(NCH):
        adj_ch = lax.dynamic_slice_in_dim(adj_matrix, ch * e_chunk, e_chunk, 0)
        ga, gv = _gather_call(atom, nvr, src5[ch], e_chunk)
        ga = ga.reshape(e_chunk, 3 * F)
        gv = gv.reshape(e_chunk, 3 * F)
        dist2 = adj_ch[:, 5].reshape(-1, F)
        feats = _sinc_call(dist2).reshape(R + 4, e_chunk)
        msgs_list.append(_combine_call(adj_ch, feats, ga, gv, wrbf_ext, e_chunk))

    nvt = jnp.moveaxis(node_vector, -1, 0)
    base = jnp.concatenate([node_scalar[None], nvt], axis=0)
    base = jnp.pad(base, ((0, 0), (0, n_pad - n), (0, 0)))
    dst5 = dst_i.reshape(NCH, N_SUBCORES, -1, IB, SCC)
    out = _scatter_call(msgs_list, base, dst5, n_pad, e_chunk)

    out_s = out[0, :n]
    out_v = jnp.stack([out[1, :n], out[2, :n], out[3, :n]], axis=-1)
    return out_s, out_v


# speedup vs baseline: 1.0785x; 1.0733x over previous
import functools

import jax
import jax.numpy as jnp
from jax import lax
from jax.experimental import pallas as pl
from jax.experimental.pallas import tpu as pltpu
from jax.experimental.pallas import tpu_sc as plsc

F = 128
R = 20
CUTOFF = 5.0

N_CORES = 2
N_SUBCORES = 16
N_WORKERS = N_CORES * N_SUBCORES

GC = 80
SCC = 40
BE = 2560
NCH = 1
BN = 512
HOP = 128
IB = 10
NB = 5


def _mlp_call(ns_pad, W1, b1r, W2, b2r):
    npad = ns_pad.shape[0]

    def body(ns_ref, w1_ref, b1_ref, w2_ref, b2_ref, out_ref):
        x = ns_ref[...]
        h = jnp.dot(x, w1_ref[...], preferred_element_type=jnp.float32,
                    precision=lax.Precision.HIGHEST) + b1_ref[...]
        h = h * jax.nn.sigmoid(h)
        out_ref[...] = jnp.dot(h, w2_ref[...], preferred_element_type=jnp.float32,
                               precision=lax.Precision.HIGHEST) + b2_ref[...]

    return pl.pallas_call(
        body,
        grid=(npad // BN,),
        in_specs=[
            pl.BlockSpec((BN, F), lambda i: (i, 0)),
            pl.BlockSpec((F, F), lambda i: (0, 0)),
            pl.BlockSpec((1, F), lambda i: (0, 0)),
            pl.BlockSpec((F, 3 * F), lambda i: (0, 0)),
            pl.BlockSpec((1, 3 * F), lambda i: (0, 0)),
        ],
        out_specs=pl.BlockSpec((BN, 3 * F), lambda i: (i, 0)),
        out_shape=jax.ShapeDtypeStruct((npad, 3 * F), jnp.float32),
    )(ns_pad, W1, b1r, W2, b2r)


def _gather_call(tab, src3, e_pad, n_pad):
    epw = e_pad // N_WORKERS
    n_chunks = epw // GC
    rows_per_tile = n_pad // N_SUBCORES
    mesh = plsc.VectorSubcoreMesh(core_axis_name="c", subcore_axis_name="s")

    @functools.partial(
        pl.kernel,
        out_type=jax.ShapeDtypeStruct((6, e_pad // GC, GC, F), jnp.float32),
        mesh=mesh,
        scratch_types=[
            pltpu.VMEM((n_chunks, GC), jnp.int32),
            pltpu.VMEM((2, GC, F), jnp.float32),
            pltpu.VMEM((16, F), jnp.float32),
            pltpu.VMEM_SHARED((n_pad, F), jnp.float32),
            pltpu.SemaphoreType.DMA,
            pltpu.SemaphoreType.DMA,
        ],
    )
    def gather_k(tab_hbm, src3_hbm, gout_hbm, idx_v, buf, hop, tab_s, sg0, sg1):
        c = lax.axis_index("c")
        s = lax.axis_index("s")
        wid = s * N_CORES + c
        sg = (sg0, sg1)
        pltpu.sync_copy(src3_hbm.at[wid], idx_v)

        for p in range(6):
            colbase = (p % 3) * F
            rowbase = (p // 3) * n_pad
            r0 = s * rows_per_tile

            def stage_hop(h, carry):
                rr = r0 + h * 16
                pltpu.sync_copy(
                    tab_hbm.at[pl.ds(rowbase + rr, 16), pl.ds(colbase, F)], hop)
                pltpu.sync_copy(hop, tab_s.at[pl.ds(rr, 16)])
                return carry

            lax.fori_loop(0, rows_per_tile // 16, stage_hop, 0)
            plsc.subcore_barrier()

            def issue(chunk, slot):
                pltpu.async_copy(tab_s.at[idx_v.at[chunk]], buf.at[slot],
                                 sg[slot])

            issue(0, 0)

            def body(g, carry):
                for j in range(2):
                    chunk = g * 2 + j

                    @pl.when(chunk + 1 < n_chunks)
                    def _():
                        issue(chunk + 1, (j + 1) % 2)

                    pltpu.make_async_copy(
                        tab_s.at[idx_v.at[chunk]], buf.at[j], sg[j]).wait()
                    pltpu.sync_copy(buf.at[j],
                                    gout_hbm.at[p, wid * n_chunks + chunk])
                return carry

            lax.fori_loop(0, n_chunks // 2, body, 0)

            if n_chunks % 2 == 1:
                chunk = n_chunks - 1
                pltpu.make_async_copy(
                    tab_s.at[idx_v.at[chunk]], buf.at[0], sg[0]).wait()
                pltpu.sync_copy(buf.at[0],
                                gout_hbm.at[p, wid * n_chunks + chunk])

            plsc.subcore_barrier()

    return gather_k(tab, src3)


def _sinc_call(dist2):
    rows = dist2.shape[0]
    bd = rows

    def body(d_ref, out_ref):
        d = d_ref[...]
        th = d * (jnp.pi / CUTOFF)
        s1 = jnp.sin(th)
        c1 = jnp.cos(th)
        cc = jnp.where(d < CUTOFF, 0.5 * (c1 + 1.0), 0.0)
        inv = cc / d
        twoc = 2.0 * c1
        out_ref[0] = s1 * inv
        s2 = twoc * s1
        out_ref[1] = s2 * inv
        prev2, prev1 = s1, s2
        for k in range(3, R + 1):
            sk = twoc * prev1 - prev2
            out_ref[k - 1] = sk * inv
            prev2, prev1 = prev1, sk
        out_ref[R] = cc
        zero = jnp.zeros_like(d)
        for k in range(R + 1, R + 4):
            out_ref[k] = zero

    return pl.pallas_call(
        body,
        grid=(rows // bd,),
        in_specs=[pl.BlockSpec((bd, F), lambda i: (i, 0))],
        out_specs=pl.BlockSpec((R + 4, bd, F), lambda i: (0, i, 0)),
        out_shape=jax.ShapeDtypeStruct((R + 4, rows, F), jnp.float32),
    )(dist2)


def _combine_call(adj, feats, g6, wrbf_ext, e_pad):
    def body(adj_ref, feats_ref, g6_ref, wrbf_ref, out_ref):
        adj_b = adj_ref[...]
        dist = adj_b[:, 5:6]
        rvec = adj_b[:, 2:5]
        rbf = lax.dot_general(feats_ref[...], wrbf_ref[...],
                              dimension_numbers=(((0,), (0,)), ((), ())),
                              preferred_element_type=jnp.float32,
                              precision=lax.Precision.HIGHEST)
        g6_b = g6_ref[...]
        s1 = g6_b[0] * rbf[:, :F]
        s2 = g6_b[1] * rbf[:, F:2 * F]
        s3 = g6_b[2] * rbf[:, 2 * F:]
        rhat = rvec / dist
        out_ref[0] = s2
        out_ref[1] = g6_b[3] * s1 + s3 * rhat[:, 0:1]
        out_ref[2] = g6_b[4] * s1 + s3 * rhat[:, 1:2]
        out_ref[3] = g6_b[5] * s1 + s3 * rhat[:, 2:3]

    return pl.pallas_call(
        body,
        grid=(e_pad // BE,),
        in_specs=[
            pl.BlockSpec((BE, 6), lambda i: (i, 0)),
            pl.BlockSpec((R + 4, BE), lambda i: (0, i)),
            pl.BlockSpec((6, BE, F), lambda i: (0, i, 0)),
            pl.BlockSpec((R + 4, 3 * F), lambda i: (0, 0)),
        ],
        out_specs=pl.BlockSpec((4, BE, F), lambda i: (0, i, 0)),
        out_shape=jax.ShapeDtypeStruct((4, e_pad, F), jnp.float32),
    )(adj, feats, g6, wrbf_ext)


def _scatter_call(msgs_list, base, dst5, n_pad, e_chunk):
    ept = e_chunk // N_SUBCORES
    n_chunks = ept // SCC
    n_groups = n_chunks // IB
    rows_per_tile = n_pad // N_SUBCORES
    n_hops = rows_per_tile // HOP
    mesh = plsc.VectorSubcoreMesh(core_axis_name="c", subcore_axis_name="s")

    @functools.partial(
        pl.kernel,
        out_type=jax.ShapeDtypeStruct((4, n_pad, F), jnp.float32),
        mesh=mesh,
        scratch_types=[
            pltpu.VMEM((IB, SCC), jnp.int32),
            pltpu.VMEM((NB * SCC, F), jnp.float32),
            pltpu.VMEM_SHARED((n_pad, F), jnp.float32),
            pltpu.SemaphoreType.DMA,
            pltpu.SemaphoreType.DMA,
            pltpu.SemaphoreType.DMA,
            pltpu.SemaphoreType.DMA,
            pltpu.SemaphoreType.DMA,
            pltpu.SemaphoreType.DMA,
            pltpu.SemaphoreType.DMA,
            pltpu.SemaphoreType.DMA,
            pltpu.SemaphoreType.DMA,
            pltpu.SemaphoreType.DMA,
        ],
    )
    def scatter_k(m0, base_hbm, dst5_hbm, out_hbm,
                  idx_v, upd_v, acc,
                  ls0, ls1, ls2, ls3, ls4, ss0, ss1, ss2, ss3, ss4):
        msgs_refs = (m0,)
        c = lax.axis_index("c")
        s = lax.axis_index("s")
        ls = (ls0, ls1, ls2, ls3, ls4)
        ss = (ss0, ss1, ss2, ss3, ss4)
        row0 = s * rows_per_tile
        stage = upd_v.at[pl.ds(0, HOP)]
        last_slot = (IB - 1) % NB

        def drain_last():
            pltpu.make_async_copy(
                upd_v.at[pl.ds(last_slot * SCC, SCC)],
                acc.at[idx_v.at[0]], ss[last_slot]).wait()

        for k in range(2):
            p = 2 * c + k
            def init_hop(h, carry):
                pltpu.sync_copy(base_hbm.at[p, pl.ds(row0 + h * HOP, HOP)],
                                stage)
                pltpu.sync_copy(stage, acc.at[pl.ds(row0 + h * HOP, HOP)])
                return carry

            lax.fori_loop(0, n_hops, init_hop, 0)
            plsc.subcore_barrier()

            for m in range(NCH):
                mh = msgs_refs[m]

                def load_src(chunk):
                    return mh.at[p, pl.ds(s * ept + chunk * SCC, SCC)]

                def issue_load(chunk, slot):
                    pltpu.async_copy(load_src(chunk),
                                     upd_v.at[pl.ds(slot * SCC, SCC)],
                                     ls[slot])

                for slot in range(NB - 1):
                    issue_load(slot, slot)

                def sc_group(g, carry):
                    @pl.when(g >= 1)
                    def _():
                        drain_last()

                    pltpu.sync_copy(dst5_hbm.at[m, s, g], idx_v)
                    for j in range(IB):
                        chunk = g * IB + j
                        slot = j % NB
                        bsl = pl.ds(slot * SCC, SCC)
                        pltpu.make_async_copy(load_src(chunk), upd_v.at[bsl],
                                              ls[slot]).wait()
                        pltpu.async_copy(upd_v.at[bsl], acc.at[idx_v.at[j]],
                                         ss[slot], add=True)
                        pslot = (j - 1) % NB
                        if j >= 1:
                            pltpu.make_async_copy(
                                upd_v.at[pl.ds(pslot * SCC, SCC)],
                                acc.at[idx_v.at[j]], ss[pslot]).wait()

                        @pl.when(chunk + NB - 1 < n_chunks)
                        def _():
                            issue_load(chunk + NB - 1, pslot)
                    return carry

                lax.fori_loop(0, n_groups, sc_group, 0)
                drain_last()

            plsc.subcore_barrier()

            def wb_hop(h, carry):
                pltpu.sync_copy(acc.at[pl.ds(row0 + h * HOP, HOP)], stage)
                pltpu.sync_copy(stage,
                                out_hbm.at[p, pl.ds(row0 + h * HOP, HOP)])
                return carry

            lax.fori_loop(0, n_hops, wb_hop, 0)
            plsc.subcore_barrier()

    return scatter_k(*msgs_list, base, dst5)


def kernel(node_scalar, node_vector, adj_matrix, W1, b1, W2, b2, Wrbf, brbf):
    n, f = node_scalar.shape
    e = adj_matrix.shape[0]
    assert f == F
    n_pad = ((n + N_SUBCORES * HOP - 1) // (N_SUBCORES * HOP)) * (N_SUBCORES * HOP)
    e_chunk = e // NCH
    assert e_chunk * NCH == e and e_chunk % (N_WORKERS * GC) == 0
    assert e_chunk % BE == 0 and e_chunk % (N_SUBCORES * SCC * IB) == 0

    src_i = adj_matrix[:, 1].astype(jnp.int32)
    dst_i = adj_matrix[:, 0].astype(jnp.int32)

    ns_pad = jnp.pad(node_scalar, ((0, n_pad - n), (0, 0)))
    atom = _mlp_call(ns_pad, W1, b1.reshape(1, F), W2, b2.reshape(1, 3 * F))

    nvr = jnp.swapaxes(node_vector, 1, 2).reshape(n, 3 * F)
    src5 = src_i.reshape(NCH, N_WORKERS, -1, GC)
    wrbf_ext = jnp.concatenate(
        [Wrbf, brbf[None, :], jnp.zeros((3, 3 * F), jnp.float32)], axis=0)
    nvr_pad = jnp.pad(nvr, ((0, n_pad - n), (0, 0)))
    tab = jnp.concatenate([atom, nvr_pad], axis=0)
    msgs_list = []
    for ch in range(NCH):
        adj_ch = lax.dynamic_slice_in_dim(adj_matrix, ch * e_chunk, e_chunk, 0)
        g6 = _gather_call(tab, src5[ch], e_chunk, n_pad).reshape(6, e_chunk, F)
        dist2 = adj_ch[:, 5].reshape(-1, F)
        feats = _sinc_call(dist2).reshape(R + 4, e_chunk)
        msgs_list.append(_combine_call(adj_ch, feats, g6, wrbf_ext, e_chunk))

    nvt = jnp.moveaxis(node_vector, -1, 0)
    base = jnp.concatenate([node_scalar[None], nvt], axis=0)
    base = jnp.pad(base, ((0, 0), (0, n_pad - n), (0, 0)))
    dst5 = dst_i.reshape(NCH, N_SUBCORES, -1, IB, SCC)
    out = _scatter_call(msgs_list, base, dst5, n_pad, e_chunk)

    out_s = out[0, :n]
    out_v = jnp.stack([out[1, :n], out[2, :n], out[3, :n]], axis=-1)
    return out_s, out_v


# speedup vs baseline: 1.1708x; 1.0856x over previous
import functools

import jax
import jax.numpy as jnp
from jax import lax
from jax.experimental import pallas as pl
from jax.experimental.pallas import tpu as pltpu
from jax.experimental.pallas import tpu_sc as plsc

F = 128
R = 20
CUTOFF = 5.0

N_CORES = 2
N_SUBCORES = 16
N_WORKERS = N_CORES * N_SUBCORES

GC = 80
SCC = 40
BE = 2560
NCH = 1
BN = 512
HOP = 128
IB = 10
NB = 5


def _mlp_call(ns_pad, W1, b1r, W2, b2r):
    npad = ns_pad.shape[0]

    def body(ns_ref, w1_ref, b1_ref, w2_ref, b2_ref, out_ref):
        x = ns_ref[...]
        h = jnp.dot(x, w1_ref[...], preferred_element_type=jnp.float32,
                    precision=lax.Precision.HIGHEST) + b1_ref[...]
        h = h * jax.nn.sigmoid(h)
        out_ref[...] = jnp.dot(h, w2_ref[...], preferred_element_type=jnp.float32,
                               precision=lax.Precision.HIGHEST) + b2_ref[...]

    return pl.pallas_call(
        body,
        grid=(npad // BN,),
        in_specs=[
            pl.BlockSpec((BN, F), lambda i: (i, 0)),
            pl.BlockSpec((F, F), lambda i: (0, 0)),
            pl.BlockSpec((1, F), lambda i: (0, 0)),
            pl.BlockSpec((F, 3 * F), lambda i: (0, 0)),
            pl.BlockSpec((1, 3 * F), lambda i: (0, 0)),
        ],
        out_specs=pl.BlockSpec((BN, 3 * F), lambda i: (i, 0)),
        out_shape=jax.ShapeDtypeStruct((npad, 3 * F), jnp.float32),
    )(ns_pad, W1, b1r, W2, b2r)


def _gather_call(tab, src3, e_pad, n_pad):
    epw = e_pad // N_WORKERS
    n_chunks = epw // GC
    rows_per_tile = n_pad // N_SUBCORES
    mesh = plsc.VectorSubcoreMesh(core_axis_name="c", subcore_axis_name="s")

    @functools.partial(
        pl.kernel,
        out_type=jax.ShapeDtypeStruct((6, e_pad // GC, GC, F), jnp.float32),
        mesh=mesh,
        scratch_types=[
            pltpu.VMEM((n_chunks, GC), jnp.int32),
            pltpu.VMEM((2, GC, F), jnp.float32),
            pltpu.VMEM_SHARED((n_pad, F), jnp.float32),
            pltpu.SemaphoreType.DMA,
            pltpu.SemaphoreType.DMA,
            pltpu.SemaphoreType.DMA,
            pltpu.SemaphoreType.DMA,
        ],
    )
    def gather_k(tab_hbm, src3_hbm, gout_hbm, idx_v, buf, tab_s,
                 sg0, sg1, so0, so1):
        c = lax.axis_index("c")
        s = lax.axis_index("s")
        wid = s * N_CORES + c
        sg = (sg0, sg1)
        so = (so0, so1)
        pltpu.sync_copy(src3_hbm.at[wid], idx_v)

        for p in range(6):
            colbase = (p % 3) * F
            rowbase = (p // 3) * n_pad
            r0 = s * rows_per_tile

            pltpu.sync_copy(
                tab_hbm.at[pl.ds(rowbase + r0, rows_per_tile),
                           pl.ds(colbase, F)],
                tab_s.at[pl.ds(r0, rows_per_tile)])
            plsc.subcore_barrier()

            def issue(chunk, slot):
                pltpu.async_copy(tab_s.at[idx_v.at[chunk]], buf.at[slot],
                                 sg[slot])

            def store_wait(slot):
                pltpu.make_async_copy(buf.at[slot], gout_hbm.at[p, 0],
                                      so[slot]).wait()

            issue(0, 0)

            def body(g, carry):
                for j in range(2):
                    chunk = g * 2 + j
                    nslot = (j + 1) % 2

                    if j == 0:
                        @pl.when(g >= 1)
                        def _():
                            store_wait(nslot)
                    else:
                        store_wait(nslot)

                    @pl.when(chunk + 1 < n_chunks)
                    def _():
                        issue(chunk + 1, nslot)

                    pltpu.make_async_copy(
                        tab_s.at[idx_v.at[chunk]], buf.at[j], sg[j]).wait()
                    pltpu.async_copy(buf.at[j],
                                     gout_hbm.at[p, wid * n_chunks + chunk],
                                     so[j])
                return carry

            lax.fori_loop(0, n_chunks // 2, body, 0)

            if n_chunks % 2 == 1:
                chunk = n_chunks - 1
                pltpu.make_async_copy(
                    tab_s.at[idx_v.at[chunk]], buf.at[0], sg[0]).wait()
                pltpu.async_copy(buf.at[0],
                                 gout_hbm.at[p, wid * n_chunks + chunk], so[0])

            if n_chunks % 2 == 1:
                store_wait(0)
            store_wait((n_chunks - 1) % 2 if n_chunks % 2 == 0 else 1)
            plsc.subcore_barrier()

    return gather_k(tab, src3)


def _sinc_call(dist2):
    rows = dist2.shape[0]
    bd = rows

    def body(d_ref, out_ref):
        d = d_ref[...]
        th = d * (jnp.pi / CUTOFF)
        s1 = jnp.sin(th)
        c1 = jnp.cos(th)
        cc = jnp.where(d < CUTOFF, 0.5 * (c1 + 1.0), 0.0)
        inv = cc / d
        twoc = 2.0 * c1
        out_ref[0] = s1 * inv
        s2 = twoc * s1
        out_ref[1] = s2 * inv
        prev2, prev1 = s1, s2
        for k in range(3, R + 1):
            sk = twoc * prev1 - prev2
            out_ref[k - 1] = sk * inv
            prev2, prev1 = prev1, sk
        out_ref[R] = cc
        zero = jnp.zeros_like(d)
        for k in range(R + 1, R + 4):
            out_ref[k] = zero

    return pl.pallas_call(
        body,
        grid=(rows // bd,),
        in_specs=[pl.BlockSpec((bd, F), lambda i: (i, 0))],
        out_specs=pl.BlockSpec((R + 4, bd, F), lambda i: (0, i, 0)),
        out_shape=jax.ShapeDtypeStruct((R + 4, rows, F), jnp.float32),
    )(dist2)


def _combine_call(adj, feats, g6, wrbf_ext, e_pad):
    def body(adj_ref, feats_ref, g6_ref, wrbf_ref, out_ref):
        adj_b = adj_ref[...]
        dist = adj_b[:, 5:6]
        rvec = adj_b[:, 2:5]
        rbf = lax.dot_general(feats_ref[...], wrbf_ref[...],
                              dimension_numbers=(((0,), (0,)), ((), ())),
                              preferred_element_type=jnp.float32,
                              precision=lax.Precision.HIGHEST)
        g6_b = g6_ref[...]
        s1 = g6_b[0] * rbf[:, :F]
        s2 = g6_b[1] * rbf[:, F:2 * F]
        s3 = g6_b[2] * rbf[:, 2 * F:]
        rhat = rvec / dist
        out_ref[0] = s2
        out_ref[1] = g6_b[3] * s1 + s3 * rhat[:, 0:1]
        out_ref[2] = g6_b[4] * s1 + s3 * rhat[:, 1:2]
        out_ref[3] = g6_b[5] * s1 + s3 * rhat[:, 2:3]

    return pl.pallas_call(
        body,
        grid=(e_pad // BE,),
        in_specs=[
            pl.BlockSpec((BE, 6), lambda i: (i, 0)),
            pl.BlockSpec((R + 4, BE), lambda i: (0, i)),
            pl.BlockSpec((6, BE, F), lambda i: (0, i, 0)),
            pl.BlockSpec((R + 4, 3 * F), lambda i: (0, 0)),
        ],
        out_specs=pl.BlockSpec((4, BE, F), lambda i: (0, i, 0)),
        out_shape=jax.ShapeDtypeStruct((4, e_pad, F), jnp.float32),
    )(adj, feats, g6, wrbf_ext)


def _scatter_call(msgs_list, base, dst5, n_pad, e_chunk):
    ept = e_chunk // N_SUBCORES
    n_chunks = ept // SCC
    n_groups = n_chunks // IB
    rows_per_tile = n_pad // N_SUBCORES
    n_hops = rows_per_tile // HOP
    mesh = plsc.VectorSubcoreMesh(core_axis_name="c", subcore_axis_name="s")

    @functools.partial(
        pl.kernel,
        out_type=jax.ShapeDtypeStruct((4, n_pad, F), jnp.float32),
        mesh=mesh,
        scratch_types=[
            pltpu.VMEM((IB, SCC), jnp.int32),
            pltpu.VMEM((NB * SCC, F), jnp.float32),
            pltpu.VMEM_SHARED((n_pad, F), jnp.float32),
            pltpu.SemaphoreType.DMA,
            pltpu.SemaphoreType.DMA,
            pltpu.SemaphoreType.DMA,
            pltpu.SemaphoreType.DMA,
            pltpu.SemaphoreType.DMA,
            pltpu.SemaphoreType.DMA,
            pltpu.SemaphoreType.DMA,
            pltpu.SemaphoreType.DMA,
            pltpu.SemaphoreType.DMA,
            pltpu.SemaphoreType.DMA,
        ],
    )
    def scatter_k(m0, base_hbm, dst5_hbm, out_hbm,
                  idx_v, upd_v, acc,
                  ls0, ls1, ls2, ls3, ls4, ss0, ss1, ss2, ss3, ss4):
        msgs_refs = (m0,)
        c = lax.axis_index("c")
        s = lax.axis_index("s")
        ls = (ls0, ls1, ls2, ls3, ls4)
        ss = (ss0, ss1, ss2, ss3, ss4)
        row0 = s * rows_per_tile
        stage = upd_v.at[pl.ds(0, HOP)]
        last_slot = (IB - 1) % NB

        def drain_last():
            pltpu.make_async_copy(
                upd_v.at[pl.ds(last_slot * SCC, SCC)],
                acc.at[idx_v.at[0]], ss[last_slot]).wait()

        for k in range(2):
            p = 2 * c + k
            def init_hop(h, carry):
                pltpu.sync_copy(base_hbm.at[p, pl.ds(row0 + h * HOP, HOP)],
                                stage)
                pltpu.sync_copy(stage, acc.at[pl.ds(row0 + h * HOP, HOP)])
                return carry

            lax.fori_loop(0, n_hops, init_hop, 0)
            plsc.subcore_barrier()

            for m in range(NCH):
                mh = msgs_refs[m]

                def load_src(chunk):
                    return mh.at[p, pl.ds(s * ept + chunk * SCC, SCC)]

                def issue_load(chunk, slot):
                    pltpu.async_copy(load_src(chunk),
                                     upd_v.at[pl.ds(slot * SCC, SCC)],
                                     ls[slot])

                for slot in range(NB - 1):
                    issue_load(slot, slot)

                def sc_group(g, carry):
                    @pl.when(g >= 1)
                    def _():
                        drain_last()

                    pltpu.sync_copy(dst5_hbm.at[m, s, g], idx_v)
                    for j in range(IB):
                        chunk = g * IB + j
                        slot = j % NB
                        bsl = pl.ds(slot * SCC, SCC)
                        pltpu.make_async_copy(load_src(chunk), upd_v.at[bsl],
                                              ls[slot]).wait()
                        pltpu.async_copy(upd_v.at[bsl], acc.at[idx_v.at[j]],
                                         ss[slot], add=True)
                        pslot = (j - 1) % NB
                        if j >= 1:
                            pltpu.make_async_copy(
                                upd_v.at[pl.ds(pslot * SCC, SCC)],
                                acc.at[idx_v.at[j]], ss[pslot]).wait()

                        @pl.when(chunk + NB - 1 < n_chunks)
                        def _():
                            issue_load(chunk + NB - 1, pslot)
                    return carry

                lax.fori_loop(0, n_groups, sc_group, 0)
                drain_last()

            plsc.subcore_barrier()

            def wb_hop(h, carry):
                pltpu.sync_copy(acc.at[pl.ds(row0 + h * HOP, HOP)], stage)
                pltpu.sync_copy(stage,
                                out_hbm.at[p, pl.ds(row0 + h * HOP, HOP)])
                return carry

            lax.fori_loop(0, n_hops, wb_hop, 0)
            plsc.subcore_barrier()

    return scatter_k(*msgs_list, base, dst5)


def kernel(node_scalar, node_vector, adj_matrix, W1, b1, W2, b2, Wrbf, brbf):
    n, f = node_scalar.shape
    e = adj_matrix.shape[0]
    assert f == F
    n_pad = ((n + N_SUBCORES * HOP - 1) // (N_SUBCORES * HOP)) * (N_SUBCORES * HOP)
    e_chunk = e // NCH
    assert e_chunk * NCH == e and e_chunk % (N_WORKERS * GC) == 0
    assert e_chunk % BE == 0 and e_chunk % (N_SUBCORES * SCC * IB) == 0

    src_i = adj_matrix[:, 1].astype(jnp.int32)
    dst_i = adj_matrix[:, 0].astype(jnp.int32)

    ns_pad = jnp.pad(node_scalar, ((0, n_pad - n), (0, 0)))
    atom = _mlp_call(ns_pad, W1, b1.reshape(1, F), W2, b2.reshape(1, 3 * F))

    nvr = jnp.swapaxes(node_vector, 1, 2).reshape(n, 3 * F)
    src5 = src_i.reshape(NCH, N_WORKERS, -1, GC)
    wrbf_ext = jnp.concatenate(
        [Wrbf, brbf[None, :], jnp.zeros((3, 3 * F), jnp.float32)], axis=0)
    nvr_pad = jnp.pad(nvr, ((0, n_pad - n), (0, 0)))
    tab = jnp.concatenate([atom, nvr_pad], axis=0)
    msgs_list = []
    for ch in range(NCH):
        adj_ch = lax.dynamic_slice_in_dim(adj_matrix, ch * e_chunk, e_chunk, 0)
        g6 = _gather_call(tab, src5[ch], e_chunk, n_pad).reshape(6, e_chunk, F)
        dist2 = adj_ch[:, 5].reshape(-1, F)
        feats = _sinc_call(dist2).reshape(R + 4, e_chunk)
        msgs_list.append(_combine_call(adj_ch, feats, g6, wrbf_ext, e_chunk))

    nvt = jnp.moveaxis(node_vector, -1, 0)
    base = jnp.concatenate([node_scalar[None], nvt], axis=0)
    base = jnp.pad(base, ((0, 0), (0, n_pad - n), (0, 0)))
    dst5 = dst_i.reshape(NCH, N_SUBCORES, -1, IB, SCC)
    out = _scatter_call(msgs_list, base, dst5, n_pad, e_chunk)

    out_s = out[0, :n]
    out_v = jnp.stack([out[1, :n], out[2, :n], out[3, :n]], axis=-1)
    return out_s, out_v
